# baseline (speedup 1.0000x reference)
import jax
import jax.numpy as jnp
from jax.experimental import pallas as pl
from jax.experimental.pallas import tpu as pltpu

N = 49152
ROWS = N // 128


def _loss_body(ts_ref, os_ref, td_ref, od_ref, out_ref):
    ts = ts_ref[...]
    os_ = os_ref[...]

    valid = jnp.not_equal(ts, -1.0)
    validf = valid.astype(jnp.float32)

    eps = 1e-7
    p = jnp.clip(os_, eps, 1.0 - eps)
    bce = -(ts * jnp.log(p) + (1.0 - ts) * jnp.log(1.0 - p))
    bce_sum = jnp.sum(jnp.where(valid, bce, 0.0))
    vcount = jnp.sum(validf)

    p_star = jnp.where(ts > 0.0, 1.0, 0.0) * validf
    a_y = jnp.zeros((ROWS, 128), jnp.float32)
    for c in range(4):
        d = jnp.abs(od_ref[:, c, :] - td_ref[:, c, :])
        a_y = a_y + jnp.where(d < 1.0, 0.5 * d * d, d - 0.5)
    reg_sum = jnp.sum(p_star * a_y)
    pcount = jnp.sum(p_star)

    a = bce_sum / jnp.maximum(vcount, 1.0)
    b = reg_sum / jnp.maximum(1e-7, pcount)
    out_ref[0, 0] = a + b


def kernel(target_deltas, target_scores, output_deltas, output_scores):
    ts = target_scores.reshape(ROWS, 128)
    os_ = output_scores.reshape(ROWS, 128)
    td = target_deltas.reshape(ROWS, 128, 4).transpose(0, 2, 1)
    od = output_deltas.reshape(ROWS, 128, 4).transpose(0, 2, 1)

    out = pl.pallas_call(
        _loss_body,
        out_shape=jax.ShapeDtypeStruct((1, 1), jnp.float32),
        out_specs=pl.BlockSpec(memory_space=pltpu.SMEM),
    )(ts, os_, td, od)
    return out[0, 0]


# speedup vs baseline: 2.1202x; 2.1202x over previous
import jax
import jax.numpy as jnp
from jax.experimental import pallas as pl
from jax.experimental.pallas import tpu as pltpu

N = 49152
ROWS = N // 128


def _loss_body(ts_ref, os_ref, td_ref, od_ref, out_ref):
    ts = ts_ref[...]
    os_ = os_ref[...]

    valid = jnp.not_equal(ts, -1.0)
    validf = valid.astype(jnp.float32)

    eps = 1e-7
    p = jnp.clip(os_, eps, 1.0 - eps)
    bce = -(ts * jnp.log(p) + (1.0 - ts) * jnp.log(1.0 - p))
    bce_sum = jnp.sum(jnp.where(valid, bce, 0.0))
    vcount = jnp.sum(validf)

    p_star = jnp.where(ts > 0.0, 1.0, 0.0) * validf
    d = jnp.abs(od_ref[...] - td_ref[...])
    sl1 = jnp.where(d < 1.0, 0.5 * d * d, d - 0.5)
    p_exp = jnp.broadcast_to(p_star[:, None, :], (ROWS, 4, 128))
    p_exp = p_exp.reshape(ROWS * 4, 128)
    reg_sum = jnp.sum(p_exp * sl1)
    pcount = jnp.sum(p_star)

    a = bce_sum / jnp.maximum(vcount, 1.0)
    b = reg_sum / jnp.maximum(1e-7, pcount)
    out_ref[0, 0] = a + b


def kernel(target_deltas, target_scores, output_deltas, output_scores):
    ts = target_scores.reshape(ROWS, 128)
    os_ = output_scores.reshape(ROWS, 128)
    td = target_deltas.reshape(ROWS, 128, 4).transpose(0, 2, 1).reshape(4 * ROWS, 128)
    od = output_deltas.reshape(ROWS, 128, 4).transpose(0, 2, 1).reshape(4 * ROWS, 128)

    out = pl.pallas_call(
        _loss_body,
        out_shape=jax.ShapeDtypeStruct((1, 1), jnp.float32),
        out_specs=pl.BlockSpec(memory_space=pltpu.SMEM),
    )(ts, os_, td, od)
    return out[0, 0]
